# Initial kernel scaffold; baseline (speedup 1.0000x reference)
import jax
import jax.numpy as jnp
from jax.experimental import pallas as pl


def kernel(frame, label):
    raise NotImplementedError("write your pallas kernel here")


# speedup vs baseline: 3.6085x; 3.6085x over previous
import functools

import jax
import jax.numpy as jnp
import numpy as np
from jax import lax
from jax.experimental import pallas as pl
from jax.experimental.pallas import tpu as pltpu
from jax.experimental.pallas import tpu_sc as plsc

FOV_UP_RAD = 90.0 / 180.0 * np.pi
FOV_DOWN_RAD = -90.0 / 180.0 * np.pi
FOV_RAD = abs(FOV_DOWN_RAD) + abs(FOV_UP_RAD)
W = 2048
H = 64
N = 131072

NC = 2
NS = 16
NW = NC * NS
PIX_PER_W = (H * W) // NW
ROWS_PER_W = H // NW * 1
CHUNK = 4096
NCHUNKS = N // CHUNK
VECS = CHUNK // 16


def _scatter_body(key_hbm, refl_hbm, depth_hbm, out_rd_hbm, out_lab_hbm,
                  key_v, refl_v, depth_v, rows_rd, rows_lab, dup_v):
    c = lax.axis_index("c")
    s = lax.axis_index("s")
    wid = s * NC + c
    lane = lax.iota(jnp.int32, 16)
    zf = jnp.zeros((16,), jnp.float32)
    zi = jnp.zeros((16,), jnp.int32)

    def zero_rd(i, carry):
        rows_rd[pl.ds(i * 16, 16)] = zf
        return carry

    lax.fori_loop(0, (2 * PIX_PER_W) // 16, zero_rd, 0)

    def zero_lab(i, carry):
        rows_lab[pl.ds(i * 16, 16)] = zi
        return carry

    lax.fori_loop(0, PIX_PER_W // 16, zero_lab, 0)

    def chunk_body(ci, carry):
        base = ci * CHUNK
        pltpu.sync_copy(key_hbm.at[pl.ds(base, CHUNK)], key_v)
        pltpu.sync_copy(refl_hbm.at[pl.ds(base, CHUNK)], refl_v)
        pltpu.sync_copy(depth_hbm.at[pl.ds(base, CHUNK)], depth_v)

        def vec_body(vi, c2):
            k = key_v[pl.ds(vi * 16, 16)]
            pix = k & 0x1FFFF
            m = (pix >> 12) == wid
            local = pix & 0xFFF
            lab = k >> 17
            plsc.store_scatter(dup_v, [local], lane, mask=m)
            got = plsc.load_gather(dup_v, [local], mask=m)
            keep = m & (got == lane)
            r = refl_v[pl.ds(vi * 16, 16)]
            d = depth_v[pl.ds(vi * 16, 16)]
            plsc.store_scatter(rows_rd, [local * 2], r, mask=keep)
            plsc.store_scatter(rows_rd, [local * 2 + 1], d, mask=keep)
            plsc.store_scatter(rows_lab, [local], lab, mask=keep)
            return c2

        lax.fori_loop(0, VECS, vec_body, 0)
        return carry

    lax.fori_loop(0, NCHUNKS, chunk_body, 0)

    pltpu.sync_copy(rows_rd, out_rd_hbm.at[pl.ds(wid * 2 * PIX_PER_W, 2 * PIX_PER_W)])
    pltpu.sync_copy(rows_lab, out_lab_hbm.at[pl.ds(wid * PIX_PER_W, PIX_PER_W)])


_scatter_call = pl.kernel(
    _scatter_body,
    out_type=(
        jax.ShapeDtypeStruct((H * W * 2,), jnp.float32),
        jax.ShapeDtypeStruct((H * W,), jnp.int32),
    ),
    mesh=plsc.VectorSubcoreMesh(
        core_axis_name="c", subcore_axis_name="s", num_cores=NC, num_subcores=NS
    ),
    scratch_types=(
        pltpu.VMEM((CHUNK,), jnp.int32),
        pltpu.VMEM((CHUNK,), jnp.float32),
        pltpu.VMEM((CHUNK,), jnp.float32),
        pltpu.VMEM((2 * PIX_PER_W,), jnp.float32),
        pltpu.VMEM((PIX_PER_W,), jnp.int32),
        pltpu.VMEM((PIX_PER_W,), jnp.int32),
    ),
    compiler_params=pltpu.CompilerParams(needs_layout_passes=False),
)


def kernel(frame, label):
    scan_xyz = frame[:, :3]
    reflectance = frame[:, 3]
    depth = jnp.linalg.norm(scan_xyz, axis=1)
    yaw = jnp.arctan2(scan_xyz[:, 1], scan_xyz[:, 0])
    pitch = jnp.arcsin(scan_xyz[:, 2] / depth)
    proj_x = 0.5 * (yaw / np.pi + 1.0)
    proj_y = (abs(FOV_UP_RAD) - pitch) / FOV_RAD
    proj_x = proj_x * W
    proj_y = proj_y * H
    proj_x = jnp.floor(proj_x)
    proj_x = jnp.clip(proj_x, 0, W - 1).astype(jnp.int32)
    proj_y = jnp.floor(proj_y)
    proj_y = jnp.clip(proj_y, 0, H - 1).astype(jnp.int32)
    key = (label << 17) | (proj_y * W + proj_x)
    out_rd, out_lab = _scatter_call(key, reflectance, depth)
    return out_rd.reshape(H, W, 2), out_lab.reshape(H, W)


# speedup vs baseline: 5.7295x; 1.5878x over previous
import functools

import jax
import jax.numpy as jnp
import numpy as np
from jax import lax
from jax.experimental import pallas as pl
from jax.experimental.pallas import tpu as pltpu
from jax.experimental.pallas import tpu_sc as plsc

FOV_UP_RAD = 90.0 / 180.0 * np.pi
FOV_DOWN_RAD = -90.0 / 180.0 * np.pi
FOV_RAD = abs(FOV_DOWN_RAD) + abs(FOV_UP_RAD)
W = 2048
H = 64
N = 131072

NC = 2
NS = 16
NW = NC * NS
PIX_PER_W = (H * W) // NW
CHUNK = 4096
NCHUNKS = N // CHUNK
VECS = CHUNK // 16
UNROLL = 4


def _scatter_body(key_hbm, refl_hbm, depth_hbm, out_rd_hbm, out_lab_hbm,
                  key_v0, key_v1, refl_v0, refl_v1, depth_v0, depth_v1,
                  rows_rd, rows_lab, sem0, sem1):
    c = lax.axis_index("c")
    s = lax.axis_index("s")
    wid = s * NC + c
    sems = (sem0, sem1)
    key_b = (key_v0, key_v1)
    refl_b = (refl_v0, refl_v1)
    depth_b = (depth_v0, depth_v1)
    zf = jnp.zeros((16,), jnp.float32)
    zi = jnp.zeros((16,), jnp.int32)

    def zero_rd(i, carry):
        rows_rd[pl.ds(i * 16, 16)] = zf
        return carry

    lax.fori_loop(0, (2 * PIX_PER_W) // 16, zero_rd, 0)

    def zero_lab(i, carry):
        rows_lab[pl.ds(i * 16, 16)] = zi
        return carry

    lax.fori_loop(0, PIX_PER_W // 16, zero_lab, 0)

    def start(ci, b):
        base = ci * CHUNK
        pltpu.async_copy(key_hbm.at[pl.ds(base, CHUNK)], key_b[b], sems[b])
        pltpu.async_copy(refl_hbm.at[pl.ds(base, CHUNK)], refl_b[b], sems[b])
        pltpu.async_copy(depth_hbm.at[pl.ds(base, CHUNK)], depth_b[b], sems[b])

    def wait(b):
        pltpu.make_async_copy(key_hbm.at[pl.ds(0, CHUNK)], key_b[b], sems[b]).wait()
        pltpu.make_async_copy(refl_hbm.at[pl.ds(0, CHUNK)], refl_b[b], sems[b]).wait()
        pltpu.make_async_copy(depth_hbm.at[pl.ds(0, CHUNK)], depth_b[b], sems[b]).wait()

    start(0, 0)
    start(1, 1)

    def pair_body(p, carry):
        for b in range(2):
            ci = p * 2 + b
            wait(b)
            kb, rb, db = key_b[b], refl_b[b], depth_b[b]

            def vec_body(vi, c2):
                for u in range(UNROLL):
                    off = (vi * UNROLL + u) * 16
                    k = kb[pl.ds(off, 16)]
                    pix = k & 0x1FFFF
                    m = (pix >> 12) == wid
                    local = pix & 0xFFF
                    lab = k >> 17
                    r = rb[pl.ds(off, 16)]
                    d = db[pl.ds(off, 16)]
                    plsc.store_scatter(rows_rd, [local * 2], r, mask=m)
                    plsc.store_scatter(rows_rd, [local * 2 + 1], d, mask=m)
                    plsc.store_scatter(rows_lab, [local], lab, mask=m)
                return c2

            lax.fori_loop(0, VECS // UNROLL, vec_body, 0)
            nxt = jnp.minimum(ci + 2, NCHUNKS - 1)
            start(nxt, b)
        return carry

    lax.fori_loop(0, NCHUNKS // 2, pair_body, 0)
    wait(0)
    wait(1)

    pltpu.sync_copy(rows_rd, out_rd_hbm.at[pl.ds(wid * 2 * PIX_PER_W, 2 * PIX_PER_W)])
    pltpu.sync_copy(rows_lab, out_lab_hbm.at[pl.ds(wid * PIX_PER_W, PIX_PER_W)])


_scatter_call = pl.kernel(
    _scatter_body,
    out_type=(
        jax.ShapeDtypeStruct((H * W * 2,), jnp.float32),
        jax.ShapeDtypeStruct((H * W,), jnp.int32),
    ),
    mesh=plsc.VectorSubcoreMesh(
        core_axis_name="c", subcore_axis_name="s", num_cores=NC, num_subcores=NS
    ),
    scratch_types=(
        pltpu.VMEM((CHUNK,), jnp.int32),
        pltpu.VMEM((CHUNK,), jnp.int32),
        pltpu.VMEM((CHUNK,), jnp.float32),
        pltpu.VMEM((CHUNK,), jnp.float32),
        pltpu.VMEM((CHUNK,), jnp.float32),
        pltpu.VMEM((CHUNK,), jnp.float32),
        pltpu.VMEM((2 * PIX_PER_W,), jnp.float32),
        pltpu.VMEM((PIX_PER_W,), jnp.int32),
        pltpu.SemaphoreType.DMA,
        pltpu.SemaphoreType.DMA,
    ),
    compiler_params=pltpu.CompilerParams(needs_layout_passes=False),
)


def kernel(frame, label):
    scan_xyz = frame[:, :3]
    reflectance = frame[:, 3]
    depth = jnp.linalg.norm(scan_xyz, axis=1)
    yaw = jnp.arctan2(scan_xyz[:, 1], scan_xyz[:, 0])
    pitch = jnp.arcsin(scan_xyz[:, 2] / depth)
    proj_x = 0.5 * (yaw / np.pi + 1.0)
    proj_y = (abs(FOV_UP_RAD) - pitch) / FOV_RAD
    proj_x = proj_x * W
    proj_y = proj_y * H
    proj_x = jnp.floor(proj_x)
    proj_x = jnp.clip(proj_x, 0, W - 1).astype(jnp.int32)
    proj_y = jnp.floor(proj_y)
    proj_y = jnp.clip(proj_y, 0, H - 1).astype(jnp.int32)
    key = (label << 17) | (proj_y * W + proj_x)
    out_rd, out_lab = _scatter_call(key, reflectance, depth)
    return out_rd.reshape(H, W, 2), out_lab.reshape(H, W)
